# baseline (speedup 1.0000x reference)
import functools

import jax
import jax.numpy as jnp
from jax import lax
from jax.experimental import pallas as pl
from jax.experimental.pallas import tpu as pltpu
from jax.experimental.pallas import tpu_sc as plsc

_LANES = 16
_NC = 2
_NS = 16
_NW = _NC * _NS


def _mm_body(x_ref, w_ref, y_ref):
    y_ref[...] = lax.dot_general(
        x_ref[...], w_ref[...],
        dimension_numbers=(((1,), (1,)), ((), ())),
        preferred_element_type=jnp.float32,
    )


def _dense_stage(x, W):
    n, _ = x.shape
    o = W.shape[0]
    return pl.pallas_call(
        _mm_body,
        out_shape=jax.ShapeDtypeStruct((n, o), jnp.float32),
    )(x, W)


def _make_sc_stage(n, e, o):
    ew = e // _NW
    groups = ew // _LANES
    mesh = plsc.VectorSubcoreMesh(core_axis_name="c", subcore_axis_name="s")

    @functools.partial(
        pl.kernel,
        mesh=mesh,
        compiler_params=pltpu.CompilerParams(needs_layout_passes=False),
        out_type=jax.ShapeDtypeStruct((e * o,), jnp.float32),
        scratch_types=[
            pltpu.VMEM((n * o,), jnp.float32),
            pltpu.VMEM((ew,), jnp.int32),
            pltpu.VMEM((ew,), jnp.int32),
        ] + [pltpu.VMEM((ew,), jnp.float32) for _ in range(o)],
    )
    def sc_edge_tanh(y_hbm, row_hbm, col_hbm, out_hbm, y_v, row_v, col_v, *out_vs):
        wid = lax.axis_index("s") * _NC + lax.axis_index("c")
        base = wid * ew
        pltpu.sync_copy(y_hbm, y_v)
        pltpu.sync_copy(row_hbm.at[pl.ds(base, ew)], row_v)
        pltpu.sync_copy(col_hbm.at[pl.ds(base, ew)], col_v)

        @plsc.parallel_loop(0, groups, unroll=16)
        def step(i):
            off = i * _LANES
            ridx = row_v[pl.ds(off, _LANES)] * o
            cidx = col_v[pl.ds(off, _LANES)] * o
            for k in range(o):
                a = (plsc.load_gather(y_v, [ridx + k])
                     + plsc.load_gather(y_v, [cidx + k]))
                out_vs[k][pl.ds(off, _LANES)] = 1.0 - 2.0 / (jnp.exp(a + a) + 1.0)
        for k in range(o):
            pltpu.sync_copy(out_vs[k], out_hbm.at[pl.ds(k * e + base, ew)])

    return sc_edge_tanh


def kernel(x, edge_index, W):
    n = x.shape[0]
    o = W.shape[0]
    e = edge_index.shape[1]
    y = _dense_stage(x, W)
    out_t = _make_sc_stage(n, e, o)(
        y.reshape(-1), edge_index[0], edge_index[1])
    return out_t.reshape(2, 2, e).transpose(2, 0, 1)


# speedup vs baseline: 1.0228x; 1.0228x over previous
import functools

import jax
import jax.numpy as jnp
from jax import lax
from jax.experimental import pallas as pl
from jax.experimental.pallas import tpu as pltpu
from jax.experimental.pallas import tpu_sc as plsc

_LANES = 16
_NC = 2
_NS = 16
_NW = _NC * _NS


def _mm_body(x_ref, w_ref, y_ref):
    y_ref[...] = lax.dot_general(
        x_ref[...], w_ref[...],
        dimension_numbers=(((1,), (1,)), ((), ())),
        preferred_element_type=jnp.float32,
    )


def _dense_stage(x, W):
    n, _ = x.shape
    o = W.shape[0]
    return pl.pallas_call(
        _mm_body,
        out_shape=jax.ShapeDtypeStruct((n, o), jnp.float32),
    )(x, W)


def _make_sc_stage(n, e, o):
    ew = e // _NW
    groups = ew // _LANES
    mesh = plsc.VectorSubcoreMesh(core_axis_name="c", subcore_axis_name="s")

    @functools.partial(
        pl.kernel,
        mesh=mesh,
        compiler_params=pltpu.CompilerParams(needs_layout_passes=False),
        out_type=jax.ShapeDtypeStruct((e * o,), jnp.float32),
        scratch_types=[
            pltpu.VMEM((n * o,), jnp.float32),
            pltpu.VMEM((ew,), jnp.int32),
            pltpu.VMEM((ew,), jnp.int32),
        ] + [pltpu.VMEM((ew,), jnp.float32) for _ in range(o)],
    )
    def sc_edge_tanh(y_hbm, row_hbm, col_hbm, out_hbm, y_v, row_v, col_v, *out_vs):
        wid = lax.axis_index("s") * _NC + lax.axis_index("c")
        base = wid * ew
        pltpu.sync_copy(y_hbm, y_v)
        pltpu.sync_copy(row_hbm.at[pl.ds(base, ew)], row_v)
        pltpu.sync_copy(col_hbm.at[pl.ds(base, ew)], col_v)

        @plsc.parallel_loop(0, groups, unroll=8)
        def step(i):
            off = i * _LANES
            ridx = row_v[pl.ds(off, _LANES)] * o
            cidx = col_v[pl.ds(off, _LANES)] * o
            for k in range(o):
                a = (plsc.load_gather(y_v, [ridx + k])
                     + plsc.load_gather(y_v, [cidx + k]))
                out_vs[k][pl.ds(off, _LANES)] = 1.0 - 2.0 / (jnp.exp(a + a) + 1.0)
        for k in range(o):
            pltpu.sync_copy(out_vs[k], out_hbm.at[pl.ds(k * e + base, ew)])

    return sc_edge_tanh


def kernel(x, edge_index, W):
    n = x.shape[0]
    o = W.shape[0]
    e = edge_index.shape[1]
    y = _dense_stage(x, W)
    out_t = _make_sc_stage(n, e, o)(y.reshape(-1), edge_index[0], edge_index[1])
    return out_t.reshape(2, 2, e).transpose(2, 0, 1)


# speedup vs baseline: 1.0811x; 1.0570x over previous
import functools

import jax
import jax.numpy as jnp
from jax import lax
from jax.experimental import pallas as pl
from jax.experimental.pallas import tpu as pltpu
from jax.experimental.pallas import tpu_sc as plsc

_LANES = 16
_NC = 2
_NS = 16
_NW = _NC * _NS
_BLK = 128


def _mm_body(x_ref, w_ref, t_ref):
    y = lax.dot_general(
        x_ref[...], w_ref[...],
        dimension_numbers=(((1,), (1,)), ((), ())),
        preferred_element_type=jnp.float32,
    )
    t_ref[...] = jnp.exp(jnp.clip(y + y, -30.0, 30.0))


def _dense_stage(x, W):
    n, _ = x.shape
    o = W.shape[0]
    return pl.pallas_call(
        _mm_body,
        out_shape=jax.ShapeDtypeStruct((n, o), jnp.float32),
    )(x, W)


def _make_sc_stage(n, e, o):
    nblk = e // _BLK
    nb_lo = nblk // _NW
    n_hi = nblk - nb_lo * _NW
    nb_hi = nb_lo + 1
    ew_hi = nb_hi * _BLK
    half = (o // 2) * e
    mesh = plsc.VectorSubcoreMesh(core_axis_name="c", subcore_axis_name="s")

    @functools.partial(
        pl.kernel,
        mesh=mesh,
        compiler_params=pltpu.CompilerParams(needs_layout_passes=False),
        out_type=jax.ShapeDtypeStruct((e * o,), jnp.float32),
        scratch_types=[
            pltpu.VMEM((n * o,), jnp.float32),
            pltpu.VMEM((ew_hi,), jnp.int32),
            pltpu.VMEM((ew_hi,), jnp.int32),
            pltpu.VMEM((nb_hi * 2 * _BLK,), jnp.float32),
            pltpu.VMEM((nb_hi * 2 * _BLK,), jnp.float32),
        ],
    )
    def sc_edge_tanh(t_hbm, row_hbm, col_hbm, out_hbm, t_v, row_v, col_v,
                     ob0, ob1):
        wid = lax.axis_index("s") * _NC + lax.axis_index("c")
        pltpu.sync_copy(t_hbm, t_v)

        def body(nb, base_blk):
            ew = nb * _BLK
            base = base_blk * _BLK
            pltpu.sync_copy(row_hbm.at[pl.ds(base, ew)], row_v.at[pl.ds(0, ew)])
            pltpu.sync_copy(col_hbm.at[pl.ds(base, ew)], col_v.at[pl.ds(0, ew)])

            @plsc.parallel_loop(0, nb * 8, unroll=8)
            def step(i):
                off = i * _LANES
                ridx = row_v[pl.ds(off, _LANES)] * o
                cidx = col_v[pl.ds(off, _LANES)] * o
                ob_off = off + (i // 8) * _BLK
                for i1, ob in ((0, ob0), (1, ob1)):
                    for i2 in range(2):
                        k = i1 * 2 + i2
                        m = (plsc.load_gather(t_v, [ridx + k])
                             * plsc.load_gather(t_v, [cidx + k]))
                        ob[pl.ds(ob_off + i2 * _BLK, _LANES)] = 1.0 - 2.0 / (m + 1.0)

            for i1, ob in ((0, ob0), (1, ob1)):
                pltpu.sync_copy(
                    ob.at[pl.ds(0, nb * 2 * _BLK)],
                    out_hbm.at[pl.ds(i1 * half + base_blk * 2 * _BLK,
                                     nb * 2 * _BLK)])

        @pl.when(wid < n_hi)
        def _hi():
            body(nb_hi, wid * nb_hi)

        @pl.when(wid >= n_hi)
        def _lo():
            body(nb_lo, n_hi * nb_hi + (wid - n_hi) * nb_lo)

    return sc_edge_tanh


def kernel(x, edge_index, W):
    n = x.shape[0]
    o = W.shape[0]
    e = edge_index.shape[1]
    t = _dense_stage(x, W)
    out_f = _make_sc_stage(n, e, o)(t.reshape(-1), edge_index[0], edge_index[1])
    nblk = e // _BLK
    return (out_f.reshape(2, nblk, 2, _BLK)
            .transpose(1, 3, 0, 2)
            .reshape(e, 2, 2))


# speedup vs baseline: 1.7692x; 1.6364x over previous
import functools

import jax
import jax.numpy as jnp
from jax import lax
from jax.experimental import pallas as pl
from jax.experimental.pallas import tpu as pltpu
from jax.experimental.pallas import tpu_sc as plsc

_LANES = 16
_NC = 2
_NS = 16
_NW = _NC * _NS
_BLK = 128


def _mm_body(x_ref, w_ref, t0_ref, t1_ref, t2_ref, t3_ref):
    z = lax.dot_general(
        w_ref[...], x_ref[...],
        dimension_numbers=(((1,), (1,)), ((), ())),
        preferred_element_type=jnp.float32,
    )
    t = jnp.exp(jnp.clip(z + z, -30.0, 30.0))
    for k, ref in enumerate((t0_ref, t1_ref, t2_ref, t3_ref)):
        ref[...] = t[k, :]


def _dense_stage(x, W):
    n, _ = x.shape
    o = W.shape[0]
    tab = jax.ShapeDtypeStruct((n,), jnp.float32)
    return pl.pallas_call(
        _mm_body,
        out_shape=tuple(tab for _ in range(o)),
    )(x, W)


def _make_sc_stage(n, e, o):
    nblk = e // _BLK
    nb_lo = nblk // _NW
    n_hi = nblk - nb_lo * _NW
    nb_hi = nb_lo + 1
    ew_hi = nb_hi * _BLK
    half = (o // 2) * e
    mesh = plsc.VectorSubcoreMesh(core_axis_name="c", subcore_axis_name="s")

    @functools.partial(
        pl.kernel,
        mesh=mesh,
        compiler_params=pltpu.CompilerParams(needs_layout_passes=False),
        out_type=jax.ShapeDtypeStruct((e * o,), jnp.float32),
        scratch_types=[
            pltpu.VMEM((n,), jnp.float32),
            pltpu.VMEM((n,), jnp.float32),
            pltpu.VMEM((n,), jnp.float32),
            pltpu.VMEM((n,), jnp.float32),
            pltpu.VMEM((ew_hi,), jnp.int32),
            pltpu.VMEM((ew_hi,), jnp.int32),
            pltpu.VMEM((nb_hi * 2 * _BLK,), jnp.float32),
            pltpu.VMEM((nb_hi * 2 * _BLK,), jnp.float32),
        ],
    )
    def sc_edge_tanh(t0_hbm, t1_hbm, t2_hbm, t3_hbm, eidx_hbm, out_hbm,
                     t0_v, t1_v, t2_v, t3_v, row_v, col_v, ob0, ob1):
        wid = lax.axis_index("s") * _NC + lax.axis_index("c")
        pltpu.sync_copy(t0_hbm, t0_v)
        pltpu.sync_copy(t1_hbm, t1_v)
        pltpu.sync_copy(t2_hbm, t2_v)
        pltpu.sync_copy(t3_hbm, t3_v)
        tabs = (t0_v, t1_v, t2_v, t3_v)

        def body(nb, base_blk):
            ew = nb * _BLK
            base = base_blk * _BLK
            pltpu.sync_copy(eidx_hbm.at[pl.ds(base, ew)],
                            row_v.at[pl.ds(0, ew)])
            pltpu.sync_copy(eidx_hbm.at[pl.ds(e + base, ew)],
                            col_v.at[pl.ds(0, ew)])

            @plsc.parallel_loop(0, nb * 8, unroll=8)
            def step(i):
                off = i * _LANES
                ridx = row_v[pl.ds(off, _LANES)]
                cidx = col_v[pl.ds(off, _LANES)]
                ob_off = off + (i // 8) * _BLK
                for i1, ob in ((0, ob0), (1, ob1)):
                    for i2 in range(2):
                        tv = tabs[i1 * 2 + i2]
                        m = (plsc.load_gather(tv, [ridx])
                             * plsc.load_gather(tv, [cidx]))
                        ob[pl.ds(ob_off + i2 * _BLK, _LANES)] = 1.0 - 2.0 / (m + 1.0)

            for i1, ob in ((0, ob0), (1, ob1)):
                pltpu.sync_copy(
                    ob.at[pl.ds(0, nb * 2 * _BLK)],
                    out_hbm.at[pl.ds(i1 * half + base_blk * 2 * _BLK,
                                     nb * 2 * _BLK)])

        @pl.when(wid < n_hi)
        def _hi():
            body(nb_hi, wid * nb_hi)

        @pl.when(wid >= n_hi)
        def _lo():
            body(nb_lo, n_hi * nb_hi + (wid - n_hi) * nb_lo)

    return sc_edge_tanh


def kernel(x, edge_index, W):
    n = x.shape[0]
    o = W.shape[0]
    e = edge_index.shape[1]
    tabs = _dense_stage(x, W)
    out_f = _make_sc_stage(n, e, o)(*tabs, edge_index.reshape(-1))
    nblk = e // _BLK
    return (out_f.reshape(2, nblk, 2, _BLK)
            .transpose(1, 3, 0, 2)
            .reshape(e, 2, 2))


# speedup vs baseline: 1.9173x; 1.0837x over previous
import functools

import jax
import jax.numpy as jnp
from jax import lax
from jax.experimental import pallas as pl
from jax.experimental.pallas import tpu as pltpu
from jax.experimental.pallas import tpu_sc as plsc

_LANES = 16
_NC = 2
_NS = 16
_NW = _NC * _NS
_BLK = 128


def _mm_body(x_ref, w_ref, e_ref, *out_refs):
    o = w_ref.shape[0]
    z = lax.dot_general(
        w_ref[...], x_ref[...],
        dimension_numbers=(((1,), (1,)), ((), ())),
        preferred_element_type=jnp.float32,
    )
    t = jnp.exp(jnp.clip(z + z, -30.0, 30.0))
    for k in range(o):
        out_refs[k][...] = t[k, :]
    out_refs[o][...] = e_ref[0, :]
    out_refs[o + 1][...] = e_ref[1, :]


def _dense_stage(x, edge_index, W):
    n, _ = x.shape
    e = edge_index.shape[1]
    o = W.shape[0]
    outs = tuple(jax.ShapeDtypeStruct((n,), jnp.float32) for _ in range(o))
    outs += (jax.ShapeDtypeStruct((e,), jnp.int32),
             jax.ShapeDtypeStruct((e,), jnp.int32))
    return pl.pallas_call(
        _mm_body,
        out_shape=outs,
    )(x, W, edge_index)


def _make_sc_stage(n, e, o):
    nblk = e // _BLK
    nb_lo = nblk // _NW
    n_hi = nblk - nb_lo * _NW
    nb_hi = nb_lo + 1
    ew_hi = nb_hi * _BLK
    half = (o // 2) * e
    mesh = plsc.VectorSubcoreMesh(core_axis_name="c", subcore_axis_name="s")

    @functools.partial(
        pl.kernel,
        mesh=mesh,
        compiler_params=pltpu.CompilerParams(needs_layout_passes=False),
        out_type=jax.ShapeDtypeStruct((e * o,), jnp.float32),
        scratch_types=[
            pltpu.VMEM((n,), jnp.float32),
            pltpu.VMEM((n,), jnp.float32),
            pltpu.VMEM((n,), jnp.float32),
            pltpu.VMEM((n,), jnp.float32),
            pltpu.VMEM((ew_hi,), jnp.int32),
            pltpu.VMEM((ew_hi,), jnp.int32),
            pltpu.VMEM((nb_hi * 2 * _BLK,), jnp.float32),
            pltpu.VMEM((nb_hi * 2 * _BLK,), jnp.float32),
        ],
    )
    def sc_edge_tanh(t0_hbm, t1_hbm, t2_hbm, t3_hbm, row_hbm, col_hbm, out_hbm,
                     t0_v, t1_v, t2_v, t3_v, row_v, col_v, ob0, ob1):
        wid = lax.axis_index("s") * _NC + lax.axis_index("c")
        pltpu.sync_copy(t0_hbm, t0_v)
        pltpu.sync_copy(t1_hbm, t1_v)
        pltpu.sync_copy(t2_hbm, t2_v)
        pltpu.sync_copy(t3_hbm, t3_v)
        tabs = (t0_v, t1_v, t2_v, t3_v)

        def body(nb, base_blk):
            ew = nb * _BLK
            base = base_blk * _BLK
            pltpu.sync_copy(row_hbm.at[pl.ds(base, ew)],
                            row_v.at[pl.ds(0, ew)])
            pltpu.sync_copy(col_hbm.at[pl.ds(base, ew)],
                            col_v.at[pl.ds(0, ew)])

            @plsc.parallel_loop(0, nb * 8, unroll=8)
            def step(i):
                off = i * _LANES
                ridx = row_v[pl.ds(off, _LANES)]
                cidx = col_v[pl.ds(off, _LANES)]
                ob_off = off + (i // 8) * _BLK
                for i1, ob in ((0, ob0), (1, ob1)):
                    for i2 in range(2):
                        tv = tabs[i1 * 2 + i2]
                        m = (plsc.load_gather(tv, [ridx])
                             * plsc.load_gather(tv, [cidx]))
                        ob[pl.ds(ob_off + i2 * _BLK, _LANES)] = 1.0 - 2.0 / (m + 1.0)

            for i1, ob in ((0, ob0), (1, ob1)):
                pltpu.sync_copy(
                    ob.at[pl.ds(0, nb * 2 * _BLK)],
                    out_hbm.at[pl.ds(i1 * half + base_blk * 2 * _BLK,
                                     nb * 2 * _BLK)])

        @pl.when(wid < n_hi)
        def _hi():
            body(nb_hi, wid * nb_hi)

        @pl.when(wid >= n_hi)
        def _lo():
            body(nb_lo, n_hi * nb_hi + (wid - n_hi) * nb_lo)

    return sc_edge_tanh


def kernel(x, edge_index, W):
    n = x.shape[0]
    o = W.shape[0]
    e = edge_index.shape[1]
    parts = _dense_stage(x, edge_index, W)
    out_f = _make_sc_stage(n, e, o)(*parts)
    nblk = e // _BLK
    return (out_f.reshape(2, nblk, 2, _BLK)
            .transpose(1, 3, 0, 2)
            .reshape(e, 2, 2))
